# Initial kernel scaffold; baseline (speedup 1.0000x reference)
import jax
import jax.numpy as jnp
from jax.experimental import pallas as pl


def kernel(init_ent_emb, init_rel_emb, edge_index, edge_type, node_id, Wu, Uu, bu, Wr, Ur, br, Wh, Uh, bh, nei_W):
    raise NotImplementedError("write your pallas kernel here")


# speedup vs baseline: 5.6303x; 5.6303x over previous
import functools

import jax
import jax.numpy as jnp
from jax import lax
from jax.experimental import pallas as pl
from jax.experimental.pallas import tpu as pltpu
from jax.experimental.pallas import tpu_sc as plsc

N = 10000
E = 320000
H = 128
R = 200
L = 2
SLOPE_NEG = (1.0 / 8.0 + 1.0 / 3.0) / 2.0

NC = 2
NS = 16
NW = NC * NS
CHUNK = 100
NCH = E // CHUNK
CPW = NCH // NW
ROWS_PER_TILE = 632
NP = NS * ROWS_PER_TILE


def _gru_body(nei, wu, uu, bu, wr, ur, br, wh, uh, bh, rel, w_out, nrel_out):
    q = nei[0]
    upd = jax.nn.sigmoid(jnp.dot(wu[0] + uu[0], q, preferred_element_type=jnp.float32) + bu[0])
    rst = jax.nn.sigmoid(jnp.dot(wr[0] + ur[0], q, preferred_element_type=jnp.float32) + br[0])
    hcap = jnp.tanh(
        jnp.dot(wh[0], q, preferred_element_type=jnp.float32)
        + jnp.dot(uh[0], rst * q, preferred_element_type=jnp.float32)
        + bh[0]
    )
    w_out[0] = (1.0 - upd) * q + upd * hcap
    nrel_out[0] = -rel[0]


def _gru(nei_W, Wu, Uu, bu, Wr, Ur, br, Wh, Uh, bh, rel_emb):
    mat_spec = pl.BlockSpec((1, H, H), lambda i: (i, 0, 0))
    rel_spec = pl.BlockSpec((1, R, H), lambda i: (i, 0, 0))
    return pl.pallas_call(
        _gru_body,
        grid=(L,),
        in_specs=[mat_spec] * 10 + [rel_spec],
        out_specs=[mat_spec, rel_spec],
        out_shape=[
            jax.ShapeDtypeStruct((L, H, H), jnp.float32),
            jax.ShapeDtypeStruct((L, R, H), jnp.float32),
        ],
    )(nei_W, Wu, Uu, bu, Wr, Ur, br, Wh, Uh, bh, rel_emb)


def _sc_accum_body(h_hbm, nrel_hbm, aux_hbm, zeros_hbm, out_hbm,
                   idx_v, hrows, rrows, acc_sh, sem_h, sem_r):
    c = lax.axis_index("c")
    s = lax.axis_index("s")
    w = s * NC + c

    pltpu.sync_copy(zeros_hbm.at[pl.ds(s * ROWS_PER_TILE, ROWS_PER_TILE)],
                    acc_sh.at[pl.ds(s * ROWS_PER_TILE, ROWS_PER_TILE)])
    plsc.subcore_barrier()

    def step(j, carry):
        row = w * CPW + j
        pltpu.sync_copy(aux_hbm.at[row], idx_v)
        gh = pltpu.async_copy(h_hbm.at[idx_v.at[0]], hrows, sem_h)
        gr = pltpu.async_copy(nrel_hbm.at[idx_v.at[1]], rrows, sem_r)
        gh.wait()
        gr.wait()
        pltpu.sync_copy(hrows, acc_sh.at[idx_v.at[2]], add=True)
        pltpu.sync_copy(rrows, acc_sh.at[idx_v.at[2]], add=True)
        return carry

    lax.fori_loop(0, CPW, step, 0)
    plsc.subcore_barrier()
    pltpu.sync_copy(acc_sh.at[pl.ds(s * ROWS_PER_TILE, ROWS_PER_TILE)],
                    out_hbm.at[c, pl.ds(s * ROWS_PER_TILE, ROWS_PER_TILE)])


@functools.partial(
    pl.kernel,
    out_type=jax.ShapeDtypeStruct((NC, NP, H), jnp.float32),
    mesh=plsc.VectorSubcoreMesh(core_axis_name="c", subcore_axis_name="s"),
    scratch_types=[
        pltpu.VMEM((3, CHUNK), jnp.int32),
        pltpu.VMEM((CHUNK, H), jnp.float32),
        pltpu.VMEM((CHUNK, H), jnp.float32),
        pltpu.VMEM_SHARED((NP, H), jnp.float32),
        pltpu.SemaphoreType.DMA,
        pltpu.SemaphoreType.DMA,
    ],
)
def _sc_accum(h_hbm, nrel_hbm, aux_hbm, zeros_hbm, out_hbm,
              idx_v, hrows, rrows, acc_sh, sem_h, sem_r):
    _sc_accum_body(h_hbm, nrel_hbm, aux_hbm, zeros_hbm, out_hbm,
                   idx_v, hrows, rrows, acc_sh, sem_h, sem_r)


BN = 1000


def _combine_body(p_ref, w_ref, o_ref):
    acc = p_ref[0] + p_ref[1]
    o = jnp.dot(acc, w_ref[...], preferred_element_type=jnp.float32)
    o_ref[...] = jnp.where(o >= 0, o, o * SLOPE_NEG)


def _combine(p, w):
    return pl.pallas_call(
        _combine_body,
        grid=(N // BN,),
        in_specs=[
            pl.BlockSpec((2, BN, H), lambda i: (0, i, 0)),
            pl.BlockSpec((H, H), lambda i: (0, 0)),
        ],
        out_specs=pl.BlockSpec((BN, H), lambda i: (i, 0)),
        out_shape=jax.ShapeDtypeStruct((N, H), jnp.float32),
    )(p, w)


def kernel(init_ent_emb, init_rel_emb, edge_index, edge_type, node_id,
           Wu, Uu, bu, Wr, Ur, br, Wh, Uh, bh, nei_W):
    h = jnp.take(init_ent_emb, node_id, axis=0)
    w_ev, nrel = _gru(nei_W, Wu, Uu, bu, Wr, Ur, br, Wh, Uh, bh, init_rel_emb)
    src = edge_index[0].reshape(NCH, CHUNK)
    dst = edge_index[1].reshape(NCH, CHUNK)
    et = edge_type.reshape(NCH, CHUNK)
    aux = jnp.stack([src, et, dst], axis=1)
    zeros = jnp.zeros((NP, H), jnp.float32)
    for l in range(L):
        p = _sc_accum(h, nrel[l], aux, zeros)
        h = _combine(p[:, :N], w_ev[l])
    return h


# speedup vs baseline: 6.1320x; 1.0891x over previous
import functools

import jax
import jax.numpy as jnp
from jax import lax
from jax.experimental import pallas as pl
from jax.experimental.pallas import tpu as pltpu
from jax.experimental.pallas import tpu_sc as plsc

N = 10000
E = 320000
H = 128
R = 200
L = 2
SLOPE_NEG = (1.0 / 8.0 + 1.0 / 3.0) / 2.0

NC = 2
NS = 16
NW = NC * NS
CHUNK = 88
EP = 321024
NCH = EP // CHUNK
CPW = NCH // NW
ROWS_PER_TILE = 632
NP = NS * ROWS_PER_TILE


def _gru_body(nei, wu, uu, bu, wr, ur, br, wh, uh, bh, rel, w_out, nrel_out):
    q = nei[0]
    upd = jax.nn.sigmoid(jnp.dot(wu[0] + uu[0], q, preferred_element_type=jnp.float32) + bu[0])
    rst = jax.nn.sigmoid(jnp.dot(wr[0] + ur[0], q, preferred_element_type=jnp.float32) + br[0])
    hcap = jnp.tanh(
        jnp.dot(wh[0], q, preferred_element_type=jnp.float32)
        + jnp.dot(uh[0], rst * q, preferred_element_type=jnp.float32)
        + bh[0]
    )
    w_out[0] = (1.0 - upd) * q + upd * hcap
    nrel_out[0] = -rel[0]


def _gru(nei_W, Wu, Uu, bu, Wr, Ur, br, Wh, Uh, bh, rel_emb):
    mat_spec = pl.BlockSpec((1, H, H), lambda i: (i, 0, 0))
    rel_spec = pl.BlockSpec((1, R, H), lambda i: (i, 0, 0))
    return pl.pallas_call(
        _gru_body,
        grid=(L,),
        in_specs=[mat_spec] * 10 + [rel_spec],
        out_specs=[mat_spec, rel_spec],
        out_shape=[
            jax.ShapeDtypeStruct((L, H, H), jnp.float32),
            jax.ShapeDtypeStruct((L, R, H), jnp.float32),
        ],
    )(nei_W, Wu, Uu, bu, Wr, Ur, br, Wh, Uh, bh, rel_emb)


def _sc_accum_body(h_hbm, nrel_hbm, aux_hbm, zeros_hbm, out_hbm,
                   idx_v, hrows, rrows, acc_sh, sems):
    c = lax.axis_index("c")
    s = lax.axis_index("s")
    w = s * NC + c
    base = w * CPW
    sem_h, sem_r, sem_sh, sem_sr = sems

    def gathers(buf, row):
        pltpu.sync_copy(aux_hbm.at[row], idx_v.at[buf])
        pltpu.async_copy(h_hbm.at[idx_v.at[buf, 0]], hrows.at[buf], sem_h[buf])
        pltpu.async_copy(nrel_hbm.at[idx_v.at[buf, 1]], rrows.at[buf], sem_r[buf])

    def wait_gathers(buf):
        pltpu.make_async_copy(h_hbm.at[idx_v.at[buf, 0]], hrows.at[buf], sem_h[buf]).wait()
        pltpu.make_async_copy(nrel_hbm.at[idx_v.at[buf, 1]], rrows.at[buf], sem_r[buf]).wait()

    def wait_scatters(buf):
        pltpu.make_async_copy(hrows.at[buf], acc_sh.at[idx_v.at[buf, 2]], sem_sh[buf]).wait()
        pltpu.make_async_copy(rrows.at[buf], acc_sh.at[idx_v.at[buf, 2]], sem_sr[buf]).wait()

    pltpu.sync_copy(zeros_hbm.at[pl.ds(s * ROWS_PER_TILE, ROWS_PER_TILE)],
                    acc_sh.at[pl.ds(s * ROWS_PER_TILE, ROWS_PER_TILE)])
    plsc.subcore_barrier()

    gathers(0, base)

    def step(j, carry):
        for b in range(2):
            ch = 2 * j + b
            nb = 1 - b
            @pl.when(ch + 1 < CPW)
            def _():
                @pl.when(ch >= 1)
                def _():
                    wait_scatters(nb)
                gathers(nb, base + ch + 1)
            wait_gathers(b)
            pltpu.async_copy(hrows.at[b], acc_sh.at[idx_v.at[b, 2]], sem_sh[b], add=True)
            pltpu.async_copy(rrows.at[b], acc_sh.at[idx_v.at[b, 2]], sem_sr[b], add=True)
        return carry

    lax.fori_loop(0, CPW // 2, step, 0)
    wait_scatters(0)
    wait_scatters(1)
    plsc.subcore_barrier()
    pltpu.sync_copy(acc_sh.at[pl.ds(s * ROWS_PER_TILE, ROWS_PER_TILE)],
                    out_hbm.at[c, pl.ds(s * ROWS_PER_TILE, ROWS_PER_TILE)])


@functools.partial(
    pl.kernel,
    out_type=jax.ShapeDtypeStruct((NC, NP, H), jnp.float32),
    mesh=plsc.VectorSubcoreMesh(core_axis_name="c", subcore_axis_name="s"),
    scratch_types=[
        pltpu.VMEM((2, 3, CHUNK), jnp.int32),
        pltpu.VMEM((2, CHUNK, H), jnp.float32),
        pltpu.VMEM((2, CHUNK, H), jnp.float32),
        pltpu.VMEM_SHARED((NP, H), jnp.float32),
        [[pltpu.SemaphoreType.DMA] * 2] * 4,
    ],
)
def _sc_accum(h_hbm, nrel_hbm, aux_hbm, zeros_hbm, out_hbm,
              idx_v, hrows, rrows, acc_sh, sems):
    _sc_accum_body(h_hbm, nrel_hbm, aux_hbm, zeros_hbm, out_hbm,
                   idx_v, hrows, rrows, acc_sh, sems)


BN = 1000


def _combine_body(p_ref, w_ref, o_ref):
    acc = p_ref[0] + p_ref[1]
    o = jnp.dot(acc, w_ref[...], preferred_element_type=jnp.float32)
    o_ref[...] = jnp.where(o >= 0, o, o * SLOPE_NEG)


def _combine(p, w):
    return pl.pallas_call(
        _combine_body,
        grid=(N // BN,),
        in_specs=[
            pl.BlockSpec((2, BN, H), lambda i: (0, i, 0)),
            pl.BlockSpec((H, H), lambda i: (0, 0)),
        ],
        out_specs=pl.BlockSpec((BN, H), lambda i: (i, 0)),
        out_shape=jax.ShapeDtypeStruct((N, H), jnp.float32),
    )(p, w)


def kernel(init_ent_emb, init_rel_emb, edge_index, edge_type, node_id,
           Wu, Uu, bu, Wr, Ur, br, Wh, Uh, bh, nei_W):
    h = jnp.take(init_ent_emb, node_id, axis=0)
    w_ev, nrel = _gru(nei_W, Wu, Uu, bu, Wr, Ur, br, Wh, Uh, bh, init_rel_emb)
    pad = EP - E
    src = jnp.concatenate([edge_index[0], jnp.zeros((pad,), jnp.int32)]).reshape(NCH, CHUNK)
    dst = jnp.concatenate([edge_index[1], jnp.full((pad,), N, jnp.int32)]).reshape(NCH, CHUNK)
    et = jnp.concatenate([edge_type, jnp.zeros((pad,), jnp.int32)]).reshape(NCH, CHUNK)
    aux = jnp.stack([src, et, dst], axis=1)
    zeros = jnp.zeros((NP, H), jnp.float32)
    for l in range(L):
        p = _sc_accum(h, nrel[l], aux, zeros)
        h = _combine(p[:, :N], w_ev[l])
    return h
